# Initial kernel scaffold; baseline (speedup 1.0000x reference)
import jax
import jax.numpy as jnp
from jax.experimental import pallas as pl


def kernel(hidden_states, router_w, w1, v1, w2):
    raise NotImplementedError("write your pallas kernel here")


# speedup vs baseline: 1.1504x; 1.1504x over previous
import functools

import jax
import jax.numpy as jnp
from jax.experimental import pallas as pl

_S = 2048
_D = 1024
_F = 2048
_E = 8
_TM = 1024


def _ffn_body(w_ref, x_ref, w1_ref, v1_ref, w2_ref, out_ref):
    e = pl.program_id(1)
    ww = w_ref[...]
    lane = jax.lax.broadcasted_iota(jnp.int32, ww.shape, 1)
    m1 = jnp.max(ww, axis=-1, keepdims=True)
    a1 = jnp.argmax(ww, axis=-1)[:, None]
    masked = jnp.where(lane == a1, -jnp.inf, ww)
    m2 = jnp.max(masked, axis=-1, keepdims=True)
    a2 = jnp.argmax(masked, axis=-1)[:, None]
    denom = m1 + m2
    scale = jnp.where(a1 == e, m1 / denom, jnp.where(a2 == e, m2 / denom, 0.0))

    xb = x_ref[...]
    x1 = jax.lax.dot_general(
        xb, w1_ref[0], (((1,), (1,)), ((), ())),
        preferred_element_type=jnp.float32)
    x2 = jax.lax.dot_general(
        xb, v1_ref[0], (((1,), (1,)), ((), ())),
        preferred_element_type=jnp.float32)
    act = (x1 * jax.lax.logistic(x1) * x2).astype(jnp.bfloat16)
    y = jnp.dot(act, w2_ref[0], preferred_element_type=jnp.float32)
    contrib = scale * y

    @pl.when(e == 0)
    def _init():
        out_ref[...] = contrib

    @pl.when(e != 0)
    def _acc():
        out_ref[...] += contrib


def kernel(hidden_states, router_w, w1, v1, w2):
    x = hidden_states.reshape(_S, _D)
    logits = jnp.matmul(x.astype(jnp.float32), router_w)
    weights = jax.nn.softmax(logits.astype(jnp.float32), axis=-1)

    xb = x.astype(jnp.bfloat16)
    w1r = w1.reshape(_E, _F, _D).astype(jnp.bfloat16)
    v1r = v1.reshape(_E, _F, _D).astype(jnp.bfloat16)
    w2r = w2.reshape(_E, _F, _D).astype(jnp.bfloat16)

    grid = (_S // _TM, _E)
    out = pl.pallas_call(
        _ffn_body,
        grid=grid,
        in_specs=[
            pl.BlockSpec((_TM, _E), lambda i, e: (i, 0)),
            pl.BlockSpec((_TM, _D), lambda i, e: (i, 0)),
            pl.BlockSpec((1, _F, _D), lambda i, e: (e, 0, 0)),
            pl.BlockSpec((1, _F, _D), lambda i, e: (e, 0, 0)),
            pl.BlockSpec((1, _F, _D), lambda i, e: (e, 0, 0)),
        ],
        out_specs=pl.BlockSpec((_TM, _D), lambda i, e: (i, 0)),
        out_shape=jax.ShapeDtypeStruct((_S, _D), jnp.float32),
    )(weights, xb, w1r, v1r, w2r)

    return (out.reshape(hidden_states.shape),
            weights.reshape(hidden_states.shape[0], _S, _E))
